# Initial kernel scaffold; baseline (speedup 1.0000x reference)
import jax
import jax.numpy as jnp
from jax.experimental import pallas as pl


def kernel(pos1, pos2, w1, b1, w2, b2, bn1_gamma, bn1_beta, bn1_mean, bn1_var, bn2_gamma, bn2_beta, bn2_mean, bn2_var):
    raise NotImplementedError("write your pallas kernel here")


# speedup vs baseline: 2.5004x; 2.5004x over previous
import jax
import jax.numpy as jnp
from jax.experimental import pallas as pl
from jax.experimental.pallas import tpu as pltpu

_P = 3
_H = 32
_EPS = 1e-5
_TN = 16384


def _emb_kernel(pos1_ref, pos2_ref, w1a_ref, w1b_ref, b1_ref, w2_ref, b2_ref,
                out_ref):
    h = jnp.dot(pos1_ref[...], w1a_ref[...],
                preferred_element_type=jnp.float32)
    h += jnp.dot(pos2_ref[...], w1b_ref[...],
                 preferred_element_type=jnp.float32)
    h = jnp.maximum(h + b1_ref[...], 0.0)
    out_ref[...] = jnp.dot(h, w2_ref[...],
                           preferred_element_type=jnp.float32) + b2_ref[...]


@jax.jit
def kernel(pos1, pos2, w1, b1, w2, b2,
           bn1_gamma, bn1_beta, bn1_mean, bn1_var,
           bn2_gamma, bn2_beta, bn2_mean, bn2_var):
    n, p = pos1.shape

    s1 = bn1_gamma / jnp.sqrt(bn1_var + _EPS)
    t1 = bn1_beta - bn1_mean * s1
    s2 = bn2_gamma / jnp.sqrt(bn2_var + _EPS)
    t2 = bn2_beta - bn2_mean * s2

    w1a, w1b, w1c = w1[:, :p], w1[:, p:2 * p], w1[:, 2 * p:]
    w1a_eff = (w1a + w1c).T * s1[None, :]
    w1b_eff = (w1b - w1c).T * s1[None, :]
    b1_eff = (b1 * s1 + t1)[None, :]
    w2_eff = w2.T * s2[None, :]
    b2_eff = (b2 * s2 + t2)[None, :]

    tn = min(_TN, n)
    grid = (pl.cdiv(n, tn),)
    return pl.pallas_call(
        _emb_kernel,
        out_shape=jax.ShapeDtypeStruct((n, _H), jnp.float32),
        grid=grid,
        in_specs=[
            pl.BlockSpec((tn, p), lambda i: (i, 0)),
            pl.BlockSpec((tn, p), lambda i: (i, 0)),
            pl.BlockSpec((p, _H), lambda i: (0, 0)),
            pl.BlockSpec((p, _H), lambda i: (0, 0)),
            pl.BlockSpec((1, _H), lambda i: (0, 0)),
            pl.BlockSpec((_H, _H), lambda i: (0, 0)),
            pl.BlockSpec((1, _H), lambda i: (0, 0)),
        ],
        out_specs=pl.BlockSpec((tn, _H), lambda i: (i, 0)),
        compiler_params=pltpu.CompilerParams(
            dimension_semantics=("parallel",)),
    )(pos1, pos2, w1a_eff, w1b_eff, b1_eff, w2_eff, b2_eff)
